# Initial kernel scaffold; baseline (speedup 1.0000x reference)
import jax
import jax.numpy as jnp
from jax.experimental import pallas as pl


def kernel(hidden_states, gate_weight, e_score_correction_bias, gate_proj, up_proj, down_proj, shared_gate, shared_up, shared_down):
    raise NotImplementedError("write your pallas kernel here")


# speedup vs baseline: 1.3246x; 1.3246x over previous
import functools

import jax
import jax.numpy as jnp
from jax import lax
from jax.experimental import pallas as pl
from jax.experimental.pallas import tpu as pltpu
from jax.experimental.pallas import tpu_sc as plsc

HIDDEN = 2048
N_EXPERTS = 64
TOP_K = 8
N_GROUP = 8
TOPK_GROUP = 4
SCALING = 2.5
INTER = 1408

_INTERPRET = False

M_TILE = 128

_NC, _NS, _L = 2, 16, 16
_NW = _NC * _NS


def _gate_body(x_ref, gw_ref, b_ref, idx_ref, w_ref):
    x = x_ref[...]
    tb = x.shape[0]
    logits = lax.dot_general(x, gw_ref[...], (((1,), (1,)), ((), ())),
                             preferred_element_type=jnp.float32)
    scores = jax.nn.sigmoid(logits)
    s_choice = scores + b_ref[...]

    col = lax.broadcasted_iota(jnp.int32, (tb, N_EXPERTS), 1)
    grp = col // (N_EXPERTS // N_GROUP)
    neg = jnp.float32(-jnp.inf)

    gsc = []
    for g in range(N_GROUP):
        m = jnp.where(grp == g, s_choice, neg)
        m1 = jnp.max(m, axis=1, keepdims=True)
        i1 = jnp.min(jnp.where(m == m1, col, N_EXPERTS), axis=1, keepdims=True)
        m2 = jnp.max(jnp.where(col == i1, neg, m), axis=1, keepdims=True)
        gsc.append(m1 + m2)
    group_scores = jnp.concatenate(gsc, axis=1)

    gcol = lax.broadcasted_iota(jnp.int32, (tb, N_GROUP), 1)
    gs = group_scores
    gmask = jnp.zeros((tb, N_GROUP), jnp.float32)
    for _ in range(TOPK_GROUP):
        m1 = jnp.max(gs, axis=1, keepdims=True)
        i1 = jnp.min(jnp.where(gs == m1, gcol, N_GROUP), axis=1, keepdims=True)
        gmask = jnp.where(gcol == i1, 1.0, gmask)
        gs = jnp.where(gcol == i1, neg, gs)

    smask = jnp.zeros((tb, N_EXPERTS), jnp.float32)
    for g in range(N_GROUP):
        smask = jnp.where(grp == g,
                          jnp.broadcast_to(gmask[:, g:g + 1], (tb, N_EXPERTS)),
                          smask)

    cur = jnp.where(smask > 0, s_choice, neg)
    idx_cols, w_cols = [], []
    for _ in range(TOP_K):
        m1 = jnp.max(cur, axis=1, keepdims=True)
        i1 = jnp.min(jnp.where(cur == m1, col, N_EXPERTS), axis=1, keepdims=True)
        wk = jnp.sum(jnp.where(col == i1, scores, 0.0), axis=1, keepdims=True)
        cur = jnp.where(col == i1, neg, cur)
        idx_cols.append(i1)
        w_cols.append(wk)
    topk_idx = jnp.concatenate(idx_cols, axis=1)
    topk_w = jnp.concatenate(w_cols, axis=1)
    denom = jnp.sum(topk_w, axis=1, keepdims=True) + 1e-20
    idx_ref[...] = topk_idx
    w_ref[...] = topk_w / denom * SCALING


def _gate_tc(x, gate_weight, bias):
    t = x.shape[0]
    tb = min(512, t)
    grid = (t // tb,)
    return pl.pallas_call(
        _gate_body,
        grid=grid,
        in_specs=[
            pl.BlockSpec((tb, HIDDEN), lambda i: (i, 0)),
            pl.BlockSpec((N_EXPERTS, HIDDEN), lambda i: (0, 0)),
            pl.BlockSpec((1, N_EXPERTS), lambda i: (0, 0)),
        ],
        out_specs=[
            pl.BlockSpec((tb, TOP_K), lambda i: (i, 0)),
            pl.BlockSpec((tb, TOP_K), lambda i: (i, 0)),
        ],
        out_shape=[
            jax.ShapeDtypeStruct((t, TOP_K), jnp.int32),
            jax.ShapeDtypeStruct((t, TOP_K), jnp.float32),
        ],
        interpret=_INTERPRET,
    )(x, gate_weight, bias.reshape(1, N_EXPERTS))


def _route(topk_idx, t):
    tk = t * TOP_K
    p_total = tk + N_EXPERTS * M_TILE
    eflat = topk_idx.reshape(-1)
    counts = jnp.zeros((N_EXPERTS,), jnp.int32).at[eflat].add(1)
    padded = ((counts + M_TILE - 1) // M_TILE) * M_TILE
    pstart = jnp.concatenate([jnp.zeros((1,), jnp.int32),
                              jnp.cumsum(padded)[:-1].astype(jnp.int32)])
    ustart = jnp.concatenate([jnp.zeros((1,), jnp.int32),
                              jnp.cumsum(counts)[:-1].astype(jnp.int32)])
    order = jnp.argsort(eflat, stable=True).astype(jnp.int32)
    es = eflat[order]
    slot = pstart[es] + jnp.arange(tk, dtype=jnp.int32) - ustart[es]
    pos = jnp.zeros((tk,), jnp.int32).at[order].set(slot)
    row_ids = (jnp.arange(p_total, dtype=jnp.int32) % t).at[slot].set(order // TOP_K)
    total_used = jnp.sum(padded).astype(jnp.int32)
    n_tiles = p_total // M_TILE
    tile_starts = jnp.arange(n_tiles, dtype=jnp.int32) * M_TILE
    seg_end = (pstart + padded).astype(jnp.int32)
    te = jnp.sum(tile_starts[:, None] >= seg_end[None, :], axis=1).astype(jnp.int32)
    te = jnp.minimum(te, N_EXPERTS - 1)
    used_tiles = total_used // M_TILE
    last_e = te[jnp.maximum(used_tiles - 1, 0)]
    tile_expert = jnp.where(tile_starts < total_used, te, last_e).astype(jnp.int32)
    return row_ids, pos, tile_expert, used_tiles.reshape(1)


def _gather_sc(x, row_ids, p_total):
    h = x.shape[1]
    pw = p_total // _NW
    ch = 32
    nch = pw // ch
    mesh = plsc.VectorSubcoreMesh(core_axis_name="c", subcore_axis_name="s")

    @functools.partial(
        pl.kernel,
        out_type=jax.ShapeDtypeStruct((p_total, h), jnp.float32),
        mesh=mesh,
        scratch_types=[
            pltpu.VMEM((ch,), jnp.int32),
            pltpu.VMEM((ch, h), jnp.float32),
            pltpu.SemaphoreType.DMA,
        ],
    )
    def _gather(x_hbm, ids_hbm, out_hbm, idx_v, rows_v, sem):
        wid = lax.axis_index("s") * _NC + lax.axis_index("c")
        base = wid * pw

        def body(i, carry):
            off = base + i * ch
            pltpu.sync_copy(ids_hbm.at[pl.ds(off, ch)], idx_v)
            pltpu.async_copy(x_hbm.at[idx_v], rows_v, sem).wait()
            pltpu.sync_copy(rows_v, out_hbm.at[pl.ds(off, ch)])
            return carry

        lax.fori_loop(0, nch, body, 0)

    return _gather(x, row_ids)


_IB = 128
_NJ = INTER // _IB


def _ffn_body(te_ref, ut_ref, x_ref, g_ref, u_ref, d_ref, y_ref):
    i = pl.program_id(0)
    j = pl.program_id(1)

    @pl.when(i < ut_ref[0])
    def _():
        x = x_ref[...]
        g = lax.dot_general(x, g_ref[0], (((1,), (1,)), ((), ())),
                            preferred_element_type=jnp.float32)
        u = lax.dot_general(x, u_ref[0], (((1,), (1,)), ((), ())),
                            preferred_element_type=jnp.float32)
        h = g * jax.nn.sigmoid(g) * u
        y = lax.dot_general(h, d_ref[0], (((1,), (1,)), ((), ())),
                            preferred_element_type=jnp.float32)

        @pl.when(j == 0)
        def _():
            y_ref[...] = y

        @pl.when(j > 0)
        def _():
            y_ref[...] = y_ref[...] + y


def _serp(i, j, nj):
    return jnp.where(i % 2 == 0, j, nj - 1 - j)


def _ffn_tc(x_sorted, gate_proj, up_proj, down_proj, tile_expert, used_tiles):
    p_total = x_sorted.shape[0]
    n_tiles = p_total // M_TILE
    grid_spec = pltpu.PrefetchScalarGridSpec(
        num_scalar_prefetch=2,
        grid=(n_tiles, _NJ),
        in_specs=[
            pl.BlockSpec((M_TILE, HIDDEN), lambda i, j, te, ut: (i, 0)),
            pl.BlockSpec((1, _IB, HIDDEN),
                         lambda i, j, te, ut: (te[i], _serp(i, j, _NJ), 0)),
            pl.BlockSpec((1, _IB, HIDDEN),
                         lambda i, j, te, ut: (te[i], _serp(i, j, _NJ), 0)),
            pl.BlockSpec((1, HIDDEN, _IB),
                         lambda i, j, te, ut: (te[i], 0, _serp(i, j, _NJ))),
        ],
        out_specs=pl.BlockSpec((M_TILE, HIDDEN), lambda i, j, te, ut: (i, 0)),
    )
    return pl.pallas_call(
        _ffn_body,
        grid_spec=grid_spec,
        out_shape=jax.ShapeDtypeStruct((p_total, HIDDEN), jnp.float32),
        compiler_params=pltpu.CompilerParams(
            dimension_semantics=("arbitrary", "arbitrary")),
        interpret=_INTERPRET,
    )(tile_expert, used_tiles, x_sorted, gate_proj, up_proj, down_proj)


def _shared_body(x_ref, g_ref, u_ref, d_ref, o_ref):
    j = pl.program_id(1)
    x = x_ref[...]
    g = lax.dot_general(x, g_ref[...], (((1,), (1,)), ((), ())),
                        preferred_element_type=jnp.float32)
    u = lax.dot_general(x, u_ref[...], (((1,), (1,)), ((), ())),
                        preferred_element_type=jnp.float32)
    h = g * jax.nn.sigmoid(g) * u
    y = lax.dot_general(h, d_ref[...], (((1,), (1,)), ((), ())),
                        preferred_element_type=jnp.float32)

    @pl.when(j == 0)
    def _():
        o_ref[...] = y

    @pl.when(j > 0)
    def _():
        o_ref[...] = o_ref[...] + y


def _shared_tc(x, shared_gate, shared_up, shared_down):
    t = x.shape[0]
    sh_inter = shared_gate.shape[0]
    n_j = sh_inter // _IB
    tb = min(1024, t)
    grid = (t // tb, n_j)
    return pl.pallas_call(
        _shared_body,
        grid=grid,
        in_specs=[
            pl.BlockSpec((tb, HIDDEN), lambda i, j: (i, 0)),
            pl.BlockSpec((_IB, HIDDEN), lambda i, j: (_serp(i, j, n_j), 0)),
            pl.BlockSpec((_IB, HIDDEN), lambda i, j: (_serp(i, j, n_j), 0)),
            pl.BlockSpec((HIDDEN, _IB), lambda i, j: (0, _serp(i, j, n_j))),
        ],
        out_specs=pl.BlockSpec((tb, HIDDEN), lambda i, j: (i, 0)),
        out_shape=jax.ShapeDtypeStruct((t, HIDDEN), jnp.float32),
        compiler_params=pltpu.CompilerParams(
            dimension_semantics=("arbitrary", "arbitrary")),
        interpret=_INTERPRET,
    )(x, shared_gate, shared_up, shared_down)


def _splat(vec, lane):
    idx = jnp.full((_L,), lane, jnp.int32)
    return lax.gather(
        vec, idx[:, None],
        dimension_numbers=lax.GatherDimensionNumbers(
            offset_dims=(), collapsed_slice_dims=(0,), start_index_map=(0,)),
        slice_sizes=(1,),
        mode=lax.GatherScatterMode.PROMISE_IN_BOUNDS)

def _combine_sc(y_sorted, pos, wflat, shared_out):
    t, h = shared_out.shape
    tw = t // _NW
    g_ch = 4
    nch = tw // g_ch
    mesh = plsc.VectorSubcoreMesh(core_axis_name="c", subcore_axis_name="s")

    @functools.partial(
        pl.kernel,
        out_type=jax.ShapeDtypeStruct((t, h), jnp.float32),
        mesh=mesh,
        scratch_types=[
            pltpu.VMEM((tw * TOP_K,), jnp.int32),
            pltpu.VMEM((tw * TOP_K,), jnp.float32),
            pltpu.VMEM((g_ch * TOP_K, h), jnp.float32),
            pltpu.VMEM((g_ch, h), jnp.float32),
            pltpu.VMEM((g_ch, h), jnp.float32),
            pltpu.SemaphoreType.DMA,
        ],
    )
    def _combine(y_hbm, pos_hbm, w_hbm, sh_hbm, out_hbm,
                 pos_v, w_v, rows_v, sh_v, out_v, sem):
        wid = lax.axis_index("s") * _NC + lax.axis_index("c")
        tbase = wid * tw
        pltpu.sync_copy(pos_hbm.at[pl.ds(tbase * TOP_K, tw * TOP_K)], pos_v)
        pltpu.sync_copy(w_hbm.at[pl.ds(tbase * TOP_K, tw * TOP_K)], w_v)

        def chunk(ci, carry):
            t0 = ci * g_ch
            pltpu.async_copy(
                y_hbm.at[pos_v.at[pl.ds(t0 * TOP_K, g_ch * TOP_K)]],
                rows_v, sem).wait()
            pltpu.sync_copy(sh_hbm.at[pl.ds(tbase + t0, g_ch)], sh_v)
            wv0 = w_v[pl.ds(t0 * TOP_K, _L)]
            wv1 = w_v[pl.ds(t0 * TOP_K + _L, _L)]
            for g in range(g_ch):
                wsp = [
                    _splat(wv0 if g * TOP_K + k < _L else wv1,
                           (g * TOP_K + k) % _L)
                    for k in range(TOP_K)
                ]

                def col(c, carry2):
                    o = c * _L
                    acc = sh_v[g, pl.ds(o, _L)]
                    for k in range(TOP_K):
                        acc = acc + wsp[k] * rows_v[g * TOP_K + k, pl.ds(o, _L)]
                    out_v[g, pl.ds(o, _L)] = acc
                    return carry2

                lax.fori_loop(0, h // _L, col, 0)
            pltpu.sync_copy(out_v, out_hbm.at[pl.ds(tbase + t0, g_ch)])
            return carry

        lax.fori_loop(0, nch, chunk, 0)

    return _combine(y_sorted, pos, wflat, shared_out)


def kernel(hidden_states, gate_weight, e_score_correction_bias, gate_proj,
           up_proj, down_proj, shared_gate, shared_up, shared_down):
    b, s, h = hidden_states.shape
    t = b * s
    x = hidden_states.reshape(t, h)

    topk_idx, topk_w = _gate_tc(x, gate_weight, e_score_correction_bias)
    row_ids, pos, tile_expert, used_tiles = _route(topk_idx, t)
    p_total = t * TOP_K + N_EXPERTS * M_TILE

    x_sorted = _gather_sc(x, row_ids, p_total)
    y_sorted = _ffn_tc(x_sorted, gate_proj, up_proj, down_proj,
                       tile_expert, used_tiles)
    shared_out = _shared_tc(x, shared_gate, shared_up, shared_down)
    out = _combine_sc(y_sorted, pos, topk_w.reshape(-1), shared_out)
    return out.reshape(b, s, h)


# speedup vs baseline: 3.0230x; 2.2822x over previous
import functools

import jax
import jax.numpy as jnp
from jax import lax
from jax.experimental import pallas as pl
from jax.experimental.pallas import tpu as pltpu
from jax.experimental.pallas import tpu_sc as plsc

HIDDEN = 2048
N_EXPERTS = 64
TOP_K = 8
N_GROUP = 8
TOPK_GROUP = 4
SCALING = 2.5
INTER = 1408

_INTERPRET = False

M_TILE = 384


def _p_total(tk):
    worst = tk + N_EXPERTS * (M_TILE - 1)
    return -(-worst // M_TILE) * M_TILE

_NC, _NS, _L = 2, 16, 16
_NW = _NC * _NS


def _gate_body(x_ref, gw_ref, b_ref, idx_ref, w_ref):
    x = x_ref[...]
    tb = x.shape[0]
    logits = lax.dot_general(x, gw_ref[...], (((1,), (1,)), ((), ())),
                             preferred_element_type=jnp.float32)
    scores = jax.nn.sigmoid(logits)
    s_choice = scores + b_ref[...]

    col = lax.broadcasted_iota(jnp.int32, (tb, N_EXPERTS), 1)
    grp = col // (N_EXPERTS // N_GROUP)
    neg = jnp.float32(-jnp.inf)

    gsc = []
    for g in range(N_GROUP):
        m = jnp.where(grp == g, s_choice, neg)
        m1 = jnp.max(m, axis=1, keepdims=True)
        i1 = jnp.min(jnp.where(m == m1, col, N_EXPERTS), axis=1, keepdims=True)
        m2 = jnp.max(jnp.where(col == i1, neg, m), axis=1, keepdims=True)
        gsc.append(m1 + m2)
    group_scores = jnp.concatenate(gsc, axis=1)

    gcol = lax.broadcasted_iota(jnp.int32, (tb, N_GROUP), 1)
    gs = group_scores
    gmask = jnp.zeros((tb, N_GROUP), jnp.float32)
    for _ in range(TOPK_GROUP):
        m1 = jnp.max(gs, axis=1, keepdims=True)
        i1 = jnp.min(jnp.where(gs == m1, gcol, N_GROUP), axis=1, keepdims=True)
        gmask = jnp.where(gcol == i1, 1.0, gmask)
        gs = jnp.where(gcol == i1, neg, gs)

    smask = jnp.zeros((tb, N_EXPERTS), jnp.float32)
    for g in range(N_GROUP):
        smask = jnp.where(grp == g,
                          jnp.broadcast_to(gmask[:, g:g + 1], (tb, N_EXPERTS)),
                          smask)

    cur = jnp.where(smask > 0, s_choice, neg)
    idx_cols, w_cols = [], []
    for _ in range(TOP_K):
        m1 = jnp.max(cur, axis=1, keepdims=True)
        i1 = jnp.min(jnp.where(cur == m1, col, N_EXPERTS), axis=1, keepdims=True)
        wk = jnp.sum(jnp.where(col == i1, scores, 0.0), axis=1, keepdims=True)
        cur = jnp.where(col == i1, neg, cur)
        idx_cols.append(i1)
        w_cols.append(wk)
    topk_idx = jnp.concatenate(idx_cols, axis=1)
    topk_w = jnp.concatenate(w_cols, axis=1)
    denom = jnp.sum(topk_w, axis=1, keepdims=True) + 1e-20
    idx_ref[...] = topk_idx
    w_ref[...] = topk_w / denom * SCALING


def _gate_tc(x, gate_weight, bias):
    t = x.shape[0]
    tb = min(512, t)
    grid = (t // tb,)
    return pl.pallas_call(
        _gate_body,
        grid=grid,
        in_specs=[
            pl.BlockSpec((tb, HIDDEN), lambda i: (i, 0)),
            pl.BlockSpec((N_EXPERTS, HIDDEN), lambda i: (0, 0)),
            pl.BlockSpec((1, N_EXPERTS), lambda i: (0, 0)),
        ],
        out_specs=[
            pl.BlockSpec((tb, TOP_K), lambda i: (i, 0)),
            pl.BlockSpec((tb, TOP_K), lambda i: (i, 0)),
        ],
        out_shape=[
            jax.ShapeDtypeStruct((t, TOP_K), jnp.int32),
            jax.ShapeDtypeStruct((t, TOP_K), jnp.float32),
        ],
        interpret=_INTERPRET,
    )(x, gate_weight, bias.reshape(1, N_EXPERTS))


def _route(topk_idx, t):
    tk = t * TOP_K
    p_total = _p_total(tk)
    eflat = topk_idx.reshape(-1)
    counts = jnp.zeros((N_EXPERTS,), jnp.int32).at[eflat].add(1)
    padded = ((counts + M_TILE - 1) // M_TILE) * M_TILE
    pstart = jnp.concatenate([jnp.zeros((1,), jnp.int32),
                              jnp.cumsum(padded)[:-1].astype(jnp.int32)])
    ustart = jnp.concatenate([jnp.zeros((1,), jnp.int32),
                              jnp.cumsum(counts)[:-1].astype(jnp.int32)])
    order = jnp.argsort(eflat, stable=True).astype(jnp.int32)
    es = eflat[order]
    slot = pstart[es] + jnp.arange(tk, dtype=jnp.int32) - ustart[es]
    pos = jnp.zeros((tk,), jnp.int32).at[order].set(slot)
    row_ids = (jnp.arange(p_total, dtype=jnp.int32) % t).at[slot].set(order // TOP_K)
    total_used = jnp.sum(padded).astype(jnp.int32)
    n_tiles = p_total // M_TILE
    tile_starts = jnp.arange(n_tiles, dtype=jnp.int32) * M_TILE
    seg_end = (pstart + padded).astype(jnp.int32)
    te = jnp.sum(tile_starts[:, None] >= seg_end[None, :], axis=1).astype(jnp.int32)
    te = jnp.minimum(te, N_EXPERTS - 1)
    used_tiles = total_used // M_TILE
    last_e = te[jnp.maximum(used_tiles - 1, 0)]
    tile_expert = jnp.where(tile_starts < total_used, te, last_e).astype(jnp.int32)
    return row_ids, pos, tile_expert, used_tiles.reshape(1)


def _gather_sc(x, row_ids, p_total):
    h = x.shape[1]
    pw = p_total // _NW
    ch = next(c for c in (56, 48, 40, 32, 24, 16, 8) if pw % c == 0)
    nch = pw // ch
    mesh = plsc.VectorSubcoreMesh(core_axis_name="c", subcore_axis_name="s")

    @functools.partial(
        pl.kernel,
        out_type=jax.ShapeDtypeStruct((p_total, h), jnp.float32),
        mesh=mesh,
        scratch_types=[
            pltpu.VMEM((ch,), jnp.int32),
            pltpu.VMEM((ch, h), jnp.float32),
            pltpu.SemaphoreType.DMA,
        ],
    )
    def _gather(x_hbm, ids_hbm, out_hbm, idx_v, rows_v, sem):
        wid = lax.axis_index("s") * _NC + lax.axis_index("c")
        base = wid * pw

        def body(i, carry):
            off = base + i * ch
            pltpu.sync_copy(ids_hbm.at[pl.ds(off, ch)], idx_v)
            pltpu.async_copy(x_hbm.at[idx_v], rows_v, sem).wait()
            pltpu.sync_copy(rows_v, out_hbm.at[pl.ds(off, ch)])
            return carry

        lax.fori_loop(0, nch, body, 0)

    return _gather(x, row_ids)


def _serp(i, j, nj):
    return jnp.where(i % 2 == 0, j, nj - 1 - j)


_C0, _C1 = 768, 640


def _dot_nt(a, b):
    return lax.dot_general(a, b, (((1,), (1,)), ((), ())),
                           preferred_element_type=jnp.float32)


def _ffn_body(te_ref, ut_ref, x_ref, gp_hbm, up_hbm, dp_hbm, y_ref,
              gw, uw, dw, sg0, sg1, su0, su1, sd0, sd1):
    i = pl.program_id(0)
    e = te_ref[i]
    prev = te_ref[jnp.maximum(i - 1, 0)]
    changed = jnp.logical_or(i == 0, e != prev)
    valid = i < ut_ref[0]

    def gcp(c0, sz, sem):
        return pltpu.make_async_copy(gp_hbm.at[e, pl.ds(c0, sz)],
                                     gw.at[pl.ds(c0, sz)], sem)

    def ucp(c0, sz, sem):
        return pltpu.make_async_copy(up_hbm.at[e, pl.ds(c0, sz)],
                                     uw.at[pl.ds(c0, sz)], sem)

    def dcp(r0, rs, sem):
        return pltpu.make_async_copy(dp_hbm.at[e, pl.ds(r0, rs)],
                                     dw.at[pl.ds(r0, rs)], sem)

    hh = HIDDEN // 2

    @pl.when(jnp.logical_and(changed, valid))
    def _():
        gcp(0, _C0, sg0).start()
        ucp(0, _C0, su0).start()
        gcp(_C0, _C1, sg1).start()
        ucp(_C0, _C1, su1).start()
        dcp(0, hh, sd0).start()
        dcp(hh, hh, sd1).start()

    @pl.when(jnp.logical_and(valid, changed))
    def _():
        x = x_ref[...]
        gcp(0, _C0, sg0).wait()
        g0 = _dot_nt(x, gw[0:_C0])
        ucp(0, _C0, su0).wait()
        u0 = _dot_nt(x, uw[0:_C0])
        h0 = g0 * jax.nn.sigmoid(g0) * u0
        gcp(_C0, _C1, sg1).wait()
        g1 = _dot_nt(x, gw[_C0:INTER])
        ucp(_C0, _C1, su1).wait()
        u1 = _dot_nt(x, uw[_C0:INTER])
        h1 = g1 * jax.nn.sigmoid(g1) * u1
        h = jnp.concatenate([h0, h1], axis=1)
        dcp(0, hh, sd0).wait()
        y_ref[:, 0:hh] = _dot_nt(h, dw[0:hh])
        dcp(hh, hh, sd1).wait()
        y_ref[:, hh:HIDDEN] = _dot_nt(h, dw[hh:HIDDEN])

    @pl.when(jnp.logical_and(valid, jnp.logical_not(changed)))
    def _():
        x = x_ref[...]
        g = _dot_nt(x, gw[...])
        u = _dot_nt(x, uw[...])
        h = g * jax.nn.sigmoid(g) * u
        y_ref[...] = _dot_nt(h, dw[...])


def _ffn_tc(x_sorted, gate_proj, up_proj, down_proj, tile_expert, used_tiles):
    p_total = x_sorted.shape[0]
    n_tiles = p_total // M_TILE
    grid_spec = pltpu.PrefetchScalarGridSpec(
        num_scalar_prefetch=2,
        grid=(n_tiles,),
        in_specs=[
            pl.BlockSpec((M_TILE, HIDDEN),
                         lambda i, te, ut: (jnp.minimum(i, ut[0] - 1), 0)),
            pl.BlockSpec(memory_space=pl.ANY),
            pl.BlockSpec(memory_space=pl.ANY),
            pl.BlockSpec(memory_space=pl.ANY),
        ],
        out_specs=pl.BlockSpec((M_TILE, HIDDEN),
                               lambda i, te, ut: (jnp.minimum(i, ut[0] - 1), 0)),
        scratch_shapes=[
            pltpu.VMEM((INTER, HIDDEN), jnp.float32),
            pltpu.VMEM((INTER, HIDDEN), jnp.float32),
            pltpu.VMEM((HIDDEN, INTER), jnp.float32),
            pltpu.SemaphoreType.DMA,
            pltpu.SemaphoreType.DMA,
            pltpu.SemaphoreType.DMA,
            pltpu.SemaphoreType.DMA,
            pltpu.SemaphoreType.DMA,
            pltpu.SemaphoreType.DMA,
        ],
    )
    return pl.pallas_call(
        _ffn_body,
        grid_spec=grid_spec,
        out_shape=jax.ShapeDtypeStruct((p_total, HIDDEN), jnp.float32),
        compiler_params=pltpu.CompilerParams(
            dimension_semantics=("arbitrary",)),
        interpret=_INTERPRET,
    )(tile_expert, used_tiles, x_sorted, gate_proj, up_proj, down_proj)


def _shared_body(x_ref, g_ref, u_ref, d_ref, o_ref):
    j = pl.program_id(1)
    x = x_ref[...]
    g = lax.dot_general(x, g_ref[...], (((1,), (1,)), ((), ())),
                        preferred_element_type=jnp.float32)
    u = lax.dot_general(x, u_ref[...], (((1,), (1,)), ((), ())),
                        preferred_element_type=jnp.float32)
    h = g * jax.nn.sigmoid(g) * u
    y = lax.dot_general(h, d_ref[...], (((1,), (1,)), ((), ())),
                        preferred_element_type=jnp.float32)

    @pl.when(j == 0)
    def _():
        o_ref[...] = y

    @pl.when(j > 0)
    def _():
        o_ref[...] = o_ref[...] + y


_IB_SH = 256


def _shared_tc(x, shared_gate, shared_up, shared_down):
    t = x.shape[0]
    sh_inter = shared_gate.shape[0]
    n_j = sh_inter // _IB_SH
    tb = min(1024, t)
    grid = (t // tb, n_j)
    return pl.pallas_call(
        _shared_body,
        grid=grid,
        in_specs=[
            pl.BlockSpec((tb, HIDDEN), lambda i, j: (i, 0)),
            pl.BlockSpec((_IB_SH, HIDDEN), lambda i, j: (_serp(i, j, n_j), 0)),
            pl.BlockSpec((_IB_SH, HIDDEN), lambda i, j: (_serp(i, j, n_j), 0)),
            pl.BlockSpec((HIDDEN, _IB_SH), lambda i, j: (0, _serp(i, j, n_j))),
        ],
        out_specs=pl.BlockSpec((tb, HIDDEN), lambda i, j: (i, 0)),
        out_shape=jax.ShapeDtypeStruct((t, HIDDEN), jnp.float32),
        compiler_params=pltpu.CompilerParams(
            dimension_semantics=("arbitrary", "arbitrary")),
        interpret=_INTERPRET,
    )(x, shared_gate, shared_up, shared_down)


def _splat(vec, lane):
    idx = jnp.full((_L,), lane, jnp.int32)
    return lax.gather(
        vec, idx[:, None],
        dimension_numbers=lax.GatherDimensionNumbers(
            offset_dims=(), collapsed_slice_dims=(0,), start_index_map=(0,)),
        slice_sizes=(1,),
        mode=lax.GatherScatterMode.PROMISE_IN_BOUNDS)

def _combine_sc(y_sorted, pos, wflat, shared_out):
    t, h = shared_out.shape
    tw = t // _NW
    g_ch = 4
    nch = tw // g_ch
    mesh = plsc.VectorSubcoreMesh(core_axis_name="c", subcore_axis_name="s")

    @functools.partial(
        pl.kernel,
        out_type=jax.ShapeDtypeStruct((t, h), jnp.float32),
        mesh=mesh,
        scratch_types=[
            pltpu.VMEM((tw * TOP_K,), jnp.int32),
            pltpu.VMEM((tw * TOP_K,), jnp.float32),
            pltpu.VMEM((g_ch * TOP_K, h), jnp.float32),
            pltpu.VMEM((g_ch, h), jnp.float32),
            pltpu.VMEM((g_ch, h), jnp.float32),
            pltpu.SemaphoreType.DMA,
        ],
    )
    def _combine(y_hbm, pos_hbm, w_hbm, sh_hbm, out_hbm,
                 pos_v, w_v, rows_v, sh_v, out_v, sem):
        wid = lax.axis_index("s") * _NC + lax.axis_index("c")
        tbase = wid * tw
        pltpu.sync_copy(pos_hbm.at[pl.ds(tbase * TOP_K, tw * TOP_K)], pos_v)
        pltpu.sync_copy(w_hbm.at[pl.ds(tbase * TOP_K, tw * TOP_K)], w_v)

        def chunk(ci, carry):
            t0 = ci * g_ch
            cp = pltpu.async_copy(
                y_hbm.at[pos_v.at[pl.ds(t0 * TOP_K, g_ch * TOP_K)]],
                rows_v, sem)
            pltpu.sync_copy(sh_hbm.at[pl.ds(tbase + t0, g_ch)], sh_v)
            cp.wait()
            wv0 = w_v[pl.ds(t0 * TOP_K, _L)]
            wv1 = w_v[pl.ds(t0 * TOP_K + _L, _L)]
            for g in range(g_ch):
                wsp = [
                    _splat(wv0 if g * TOP_K + k < _L else wv1,
                           (g * TOP_K + k) % _L)
                    for k in range(TOP_K)
                ]

                def col(c, carry2):
                    o = c * _L
                    acc = sh_v[g, pl.ds(o, _L)]
                    for k in range(TOP_K):
                        acc = acc + wsp[k] * rows_v[g * TOP_K + k, pl.ds(o, _L)]
                    out_v[g, pl.ds(o, _L)] = acc
                    return carry2

                lax.fori_loop(0, h // _L, col, 0)
            pltpu.sync_copy(out_v, out_hbm.at[pl.ds(tbase + t0, g_ch)])
            return carry

        lax.fori_loop(0, nch, chunk, 0)

    return _combine(y_sorted, pos, wflat, shared_out)


def kernel(hidden_states, gate_weight, e_score_correction_bias, gate_proj,
           up_proj, down_proj, shared_gate, shared_up, shared_down):
    b, s, h = hidden_states.shape
    t = b * s
    x = hidden_states.reshape(t, h)

    topk_idx, topk_w = _gate_tc(x, gate_weight, e_score_correction_bias)
    row_ids, pos, tile_expert, used_tiles = _route(topk_idx, t)
    p_total = _p_total(t * TOP_K)

    shared_out = _shared_tc(x, shared_gate, shared_up, shared_down)
    x_sorted = _gather_sc(x, row_ids, p_total)
    y_sorted = _ffn_tc(x_sorted, gate_proj, up_proj, down_proj,
                       tile_expert, used_tiles)
    out = _combine_sc(y_sorted, pos, topk_w.reshape(-1), shared_out)
    return out.reshape(b, s, h)
